# Initial kernel scaffold; baseline (speedup 1.0000x reference)
import jax
import jax.numpy as jnp
from jax.experimental import pallas as pl


def kernel(attr, dep_table, sid_table, eid_table, W1, b1, W2, b2):
    raise NotImplementedError("write your pallas kernel here")


# speedup vs baseline: 3.1769x; 3.1769x over previous
import functools

import jax
import jax.numpy as jnp
from jax import lax
from jax.experimental import pallas as pl
from jax.experimental.pallas import tpu as pltpu
from jax.experimental.pallas import tpu_sc as plsc

B = 16384
HIDDEN = 256
EMBED = 128

SEG_OFF = (0, 288, 552)
F_ROWS = 816

NC = 2
NS = 16
LANES = 16
NW = NC * NS
BPW = B // NW
CHUNK = 128
NCHUNK = BPW // CHUNK


def _fuse_tables_kernel(t_ref, w1_ref, b1_ref, f_ref):
    f_ref[0:288, :] = (
        jnp.dot(t_ref[0:288, :], w1_ref[0:256, :],
                preferred_element_type=jnp.float32) + b1_ref[:]
    )
    f_ref[288:552, :] = jnp.dot(
        t_ref[288:552, :], w1_ref[256:512, :],
        preferred_element_type=jnp.float32)
    f_ref[552:816, :] = jnp.dot(
        t_ref[552:816, :], w1_ref[512:768, :],
        preferred_element_type=jnp.float32)


def _sc_embed_sum(f_hbm, idx_hbm, out_hbm, idx0, idx1, idx2, g0, g1, g2, sem):
    wid = lax.axis_index("s") * NC + lax.axis_index("c")
    base = wid * BPW
    pltpu.sync_copy(idx_hbm.at[pl.ds(0 * B + base, BPW)], idx0)
    pltpu.sync_copy(idx_hbm.at[pl.ds(1 * B + base, BPW)], idx1)
    pltpu.sync_copy(idx_hbm.at[pl.ds(2 * B + base, BPW)], idx2)

    def do_chunk(c):
        cp0 = pltpu.async_copy(f_hbm.at[idx0.at[pl.ds(c * CHUNK, CHUNK)]],
                               g0, sem)
        cp1 = pltpu.async_copy(f_hbm.at[idx1.at[pl.ds(c * CHUNK, CHUNK)]],
                               g1, sem)
        cp2 = pltpu.async_copy(f_hbm.at[idx2.at[pl.ds(c * CHUNK, CHUNK)]],
                               g2, sem)
        cp0.wait()
        cp1.wait()
        cp2.wait()

        def row_body(r, _):
            for j in range(EMBED // LANES):
                sl = pl.ds(j * LANES, LANES)
                g0[r, sl] = jnp.maximum(g0[r, sl] + g1[r, sl] + g2[r, sl],
                                        0.0)
            return 0

        lax.fori_loop(0, CHUNK, row_body, 0)
        pltpu.sync_copy(g0, out_hbm.at[pl.ds(base + c * CHUNK, CHUNK)])

    for c in range(NCHUNK):
        do_chunk(c)


def _mlp2_kernel(x_ref, w2_ref, b2_ref, o_ref):
    o_ref[:] = (
        jnp.dot(x_ref[:], w2_ref[:], preferred_element_type=jnp.float32)
        + b2_ref[:]
    )


def kernel(attr, dep_table, sid_table, eid_table, W1, b1, W2, b2):
    f32 = jnp.float32
    pad7 = jnp.zeros((7, HIDDEN), f32)
    tables = jnp.concatenate(
        [dep_table, sid_table, pad7, eid_table, pad7], axis=0)

    F = pl.pallas_call(
        _fuse_tables_kernel,
        out_shape=jax.ShapeDtypeStruct((F_ROWS, EMBED), f32),
    )(tables, W1, b1.reshape(1, EMBED))

    off = jnp.array(SEG_OFF, jnp.int32)
    idx = (attr + off[None, :]).T.reshape(3 * B)

    mesh = plsc.VectorSubcoreMesh(core_axis_name="c", subcore_axis_name="s")
    sc_call = functools.partial(
        pl.kernel,
        mesh=mesh,
        out_type=jax.ShapeDtypeStruct((B, EMBED), f32),
        scratch_types=[
            pltpu.VMEM((BPW,), jnp.int32),
            pltpu.VMEM((BPW,), jnp.int32),
            pltpu.VMEM((BPW,), jnp.int32),
            pltpu.VMEM((CHUNK, EMBED), f32),
            pltpu.VMEM((CHUNK, EMBED), f32),
            pltpu.VMEM((CHUNK, EMBED), f32),
            pltpu.SemaphoreType.DMA,
        ],
    )(_sc_embed_sum)
    pre = sc_call(F, idx)

    BM = 2048
    out = pl.pallas_call(
        _mlp2_kernel,
        grid=(B // BM,),
        in_specs=[
            pl.BlockSpec((BM, EMBED), lambda i: (i, 0)),
            pl.BlockSpec((EMBED, EMBED), lambda i: (0, 0)),
            pl.BlockSpec((1, EMBED), lambda i: (0, 0)),
        ],
        out_specs=pl.BlockSpec((BM, EMBED), lambda i: (i, 0)),
        out_shape=jax.ShapeDtypeStruct((B, EMBED), f32),
    )(pre, W2, b2.reshape(1, EMBED))
    return out


# speedup vs baseline: 3.6916x; 1.1620x over previous
import functools

import jax
import jax.numpy as jnp
from jax import lax
from jax.experimental import pallas as pl
from jax.experimental.pallas import tpu as pltpu
from jax.experimental.pallas import tpu_sc as plsc

B = 16384
HIDDEN = 256
EMBED = 128

SEG_OFF = (0, 288, 552)
F_ROWS = 816

NC = 2
NS = 16
LANES = 16
NW = NC * NS
BPW = B // NW
CHUNK = 128
NCHUNK = BPW // CHUNK


def _fuse_tables_kernel(dep_ref, sid_ref, eid_ref, w1_ref, b1_ref, f_ref):
    f_ref[0:288, :] = (
        jnp.dot(dep_ref[:], w1_ref[0:256, :],
                preferred_element_type=jnp.float32) + b1_ref[:]
    )
    f_ref[288:545, :] = jnp.dot(
        sid_ref[:], w1_ref[256:512, :], preferred_element_type=jnp.float32)
    f_ref[552:809, :] = jnp.dot(
        eid_ref[:], w1_ref[512:768, :], preferred_element_type=jnp.float32)


def _sc_embed_sum(f_hbm, idx_hbm, out_hbm,
                  idx0, idx1, idx2,
                  g0a, g1a, g2a, g0b, g1b, g2b,
                  sem_a, sem_b, osem_a, osem_b):
    wid = lax.axis_index("s") * NC + lax.axis_index("c")
    base = wid * BPW
    pltpu.sync_copy(idx_hbm.at[pl.ds(0 * B + base, BPW)], idx0)
    pltpu.sync_copy(idx_hbm.at[pl.ds(1 * B + base, BPW)], idx1)
    pltpu.sync_copy(idx_hbm.at[pl.ds(2 * B + base, BPW)], idx2)

    bufsets = ((g0a, g1a, g2a, sem_a, osem_a),
               (g0b, g1b, g2b, sem_b, osem_b))

    def fire(c):
        s0, s1, s2, sem, _ = bufsets[c % 2]
        sl = pl.ds(c * CHUNK, CHUNK)
        return (
            pltpu.async_copy(f_hbm.at[idx0.at[sl]], s0, sem),
            pltpu.async_copy(f_hbm.at[idx1.at[sl]], s1, sem),
            pltpu.async_copy(f_hbm.at[idx2.at[sl]], s2, sem),
        )

    cps = fire(0)
    out_cps = [None, None]
    for c in range(NCHUNK):
        s0, s1, s2, _, osem = bufsets[c % 2]
        nxt = fire(c + 1) if c + 1 < NCHUNK else None
        for cp in cps:
            cp.wait()
        if out_cps[c % 2] is not None:
            out_cps[c % 2].wait()

        def row_body(r, _):
            for j in range(EMBED // LANES):
                sl = pl.ds(j * LANES, LANES)
                s0[r, sl] = jnp.maximum(s0[r, sl] + s1[r, sl] + s2[r, sl],
                                        0.0)
            return 0

        lax.fori_loop(0, CHUNK, row_body, 0)
        out_cps[c % 2] = pltpu.async_copy(
            s0, out_hbm.at[pl.ds(base + c * CHUNK, CHUNK)], osem)
        cps = nxt
    for cp in out_cps:
        if cp is not None:
            cp.wait()


def _mlp2_kernel(x_ref, w2_ref, b2_ref, o_ref):
    o_ref[:] = (
        jnp.dot(x_ref[:], w2_ref[:], preferred_element_type=jnp.float32)
        + b2_ref[:]
    )


def kernel(attr, dep_table, sid_table, eid_table, W1, b1, W2, b2):
    f32 = jnp.float32

    F = pl.pallas_call(
        _fuse_tables_kernel,
        out_shape=jax.ShapeDtypeStruct((F_ROWS, EMBED), f32),
    )(dep_table, sid_table, eid_table, W1, b1.reshape(1, EMBED))

    off = jnp.array(SEG_OFF, jnp.int32)
    idx = (attr + off[None, :]).T.reshape(3 * B)

    mesh = plsc.VectorSubcoreMesh(core_axis_name="c", subcore_axis_name="s")
    sc_call = functools.partial(
        pl.kernel,
        mesh=mesh,
        out_type=jax.ShapeDtypeStruct((B, EMBED), f32),
        scratch_types=[
            pltpu.VMEM((BPW,), jnp.int32),
            pltpu.VMEM((BPW,), jnp.int32),
            pltpu.VMEM((BPW,), jnp.int32),
            pltpu.VMEM((CHUNK, EMBED), f32),
            pltpu.VMEM((CHUNK, EMBED), f32),
            pltpu.VMEM((CHUNK, EMBED), f32),
            pltpu.VMEM((CHUNK, EMBED), f32),
            pltpu.VMEM((CHUNK, EMBED), f32),
            pltpu.VMEM((CHUNK, EMBED), f32),
            pltpu.SemaphoreType.DMA,
            pltpu.SemaphoreType.DMA,
            pltpu.SemaphoreType.DMA,
            pltpu.SemaphoreType.DMA,
        ],
    )(_sc_embed_sum)
    pre = sc_call(F, idx)

    BM = 2048
    out = pl.pallas_call(
        _mlp2_kernel,
        grid=(B // BM,),
        in_specs=[
            pl.BlockSpec((BM, EMBED), lambda i: (i, 0)),
            pl.BlockSpec((EMBED, EMBED), lambda i: (0, 0)),
            pl.BlockSpec((1, EMBED), lambda i: (0, 0)),
        ],
        out_specs=pl.BlockSpec((BM, EMBED), lambda i: (i, 0)),
        out_shape=jax.ShapeDtypeStruct((B, EMBED), f32),
    )(pre, W2, b2.reshape(1, EMBED))
    return out
